# Initial kernel scaffold; baseline (speedup 1.0000x reference)
import functools

import jax
import jax.numpy as jnp
from jax import lax
from jax.experimental import pallas as pl
from jax.experimental.pallas import tpu as pltpu
from jax.experimental.pallas import tpu_sc as plsc

N = 10000
D_IN = 128
D_HID = 128
D_EMB = 64
E = 320000

NC = 2
NS = 16
NW = NC * NS
K = 80
EPW = E // NW
NCHUNK = EPW // K
RPT = N // NS
ZROWS = 125


def _make_sc_agg(dp):
    mesh = plsc.VectorSubcoreMesh(core_axis_name="c", subcore_axis_name="s")

    @functools.partial(
        pl.kernel,
        mesh=mesh,
        out_type=jax.ShapeDtypeStruct((NC, N, dp), jnp.float32),
        scratch_types=[
            pltpu.VMEM((K,), jnp.int32),
            pltpu.VMEM((K,), jnp.int32),
            pltpu.VMEM((K, dp), jnp.float32),
            pltpu.VMEM((ZROWS, dp), jnp.float32),
            pltpu.VMEM_SHARED((N, dp), jnp.float32),
            pltpu.SemaphoreType.DMA,
        ],
    )
    def agg(y_hbm, src_hbm, dst_hbm, zero_hbm, out_hbm,
            src_v, dst_v, rows_v, zbuf_v, acc_sh, sem):
        c = lax.axis_index("c")
        s = lax.axis_index("s")
        wid = c * NS + s

        pltpu.sync_copy(zero_hbm, zbuf_v)
        for kk in range(RPT // ZROWS):
            pltpu.sync_copy(
                zbuf_v, acc_sh.at[pl.ds(s * RPT + kk * ZROWS, ZROWS)])
        plsc.subcore_barrier()

        def body(g, carry):
            pltpu.sync_copy(src_hbm.at[wid, g], src_v)
            pltpu.sync_copy(dst_hbm.at[wid, g], dst_v)
            pltpu.async_copy(y_hbm.at[src_v], rows_v, sem).wait()
            pltpu.sync_copy(rows_v, acc_sh.at[dst_v], add=True)
            return carry

        lax.fori_loop(0, NCHUNK, body, 0)
        plsc.subcore_barrier()

        for kk in range(RPT // ZROWS):
            r0 = s * RPT + kk * ZROWS
            pltpu.sync_copy(acc_sh.at[pl.ds(r0, ZROWS)], zbuf_v)
            pltpu.sync_copy(zbuf_v, out_hbm.at[c, pl.ds(r0, ZROWS)])

    return agg


_sc_agg_144 = _make_sc_agg(D_HID + 16)
_sc_agg_128 = _make_sc_agg(D_HID)
_sc_agg_64 = _make_sc_agg(D_EMB)


def _tc_y1(x, wl):
    def body(x_ref, w_ref, o_ref):
        y = jnp.dot(x_ref[...], w_ref[...],
                    preferred_element_type=jnp.float32)
        lane = lax.broadcasted_iota(jnp.int32, (N, 16), 1)
        pad = jnp.where(lane == 0, 1.0, 0.0).astype(jnp.float32)
        o_ref[...] = jnp.concatenate([y, pad], axis=1)

    return pl.pallas_call(
        body,
        out_shape=jax.ShapeDtypeStruct((N, D_HID + 16), jnp.float32),
    )(x, wl)


def _tc_combine1(acc, x, wr, b, wl2):
    def body(acc_ref, x_ref, wr_ref, b_ref, wl2_ref, h_ref, y_ref, inv_ref):
        agg = acc_ref[0] + acc_ref[1]
        deg = agg[:, D_HID:D_HID + 1]
        inv = 1.0 / jnp.maximum(deg, 1.0)
        h = jnp.maximum(
            agg[:, :D_HID] * inv + b_ref[...]
            + jnp.dot(x_ref[...], wr_ref[...],
                      preferred_element_type=jnp.float32),
            0.0)
        h_ref[...] = h
        y_ref[...] = jnp.dot(h, wl2_ref[...],
                             preferred_element_type=jnp.float32)
        inv_ref[...] = inv

    return pl.pallas_call(
        body,
        out_shape=(
            jax.ShapeDtypeStruct((N, D_HID), jnp.float32),
            jax.ShapeDtypeStruct((N, D_HID), jnp.float32),
            jax.ShapeDtypeStruct((N, 1), jnp.float32),
        ),
    )(acc, x, wr, b, wl2)


def _tc_combine2(acc, inv, h, wr, b, wl3):
    def body(acc_ref, inv_ref, h_ref, wr_ref, b_ref, wl3_ref,
             ho_ref, y_ref):
        agg = acc_ref[0] + acc_ref[1]
        h2 = jnp.maximum(
            agg * inv_ref[...] + b_ref[...]
            + jnp.dot(h_ref[...], wr_ref[...],
                      preferred_element_type=jnp.float32),
            0.0)
        ho_ref[...] = h2
        y_ref[...] = jnp.dot(h2, wl3_ref[...],
                             preferred_element_type=jnp.float32)

    return pl.pallas_call(
        body,
        out_shape=(
            jax.ShapeDtypeStruct((N, D_HID), jnp.float32),
            jax.ShapeDtypeStruct((N, D_EMB), jnp.float32),
        ),
    )(acc, inv, h, wr, b, wl3)


def _tc_final(acc, inv, h, wr, b):
    def body(acc_ref, inv_ref, h_ref, wr_ref, b_ref, o_ref):
        agg = acc_ref[0] + acc_ref[1]
        o_ref[...] = (agg * inv_ref[...] + b_ref[...]
                      + jnp.dot(h_ref[...], wr_ref[...],
                                preferred_element_type=jnp.float32))

    return pl.pallas_call(
        body,
        out_shape=jax.ShapeDtypeStruct((N, D_EMB), jnp.float32),
    )(acc, inv, h, wr, b)


def kernel(x, edge_index, W1l, b1, W1r, W2l, b2, W2r, W3l, b3, W3r):
    ei = edge_index.astype(jnp.int32)
    src3 = ei[0].reshape(NW, NCHUNK, K)
    dst3 = ei[1].reshape(NW, NCHUNK, K)
    z144 = jnp.zeros((ZROWS, D_HID + 16), jnp.float32)
    z128 = jnp.zeros((ZROWS, D_HID), jnp.float32)
    z64 = jnp.zeros((ZROWS, D_EMB), jnp.float32)

    y1 = _tc_y1(x, W1l)
    acc1 = _sc_agg_144(y1, src3, dst3, z144)
    h1, y2, inv = _tc_combine1(acc1, x, W1r, b1.reshape(1, -1), W2l)
    acc2 = _sc_agg_128(y2, src3, dst3, z128)
    h2, y3 = _tc_combine2(acc2, inv, h1, W2r, b2.reshape(1, -1), W3l)
    acc3 = _sc_agg_64(y3, src3, dst3, z64)
    return _tc_final(acc3, inv, h2, W3r, b3.reshape(1, -1))


# speedup vs baseline: 5.1631x; 5.1631x over previous
import functools

import jax
import jax.numpy as jnp
from jax import lax
from jax.experimental import pallas as pl
from jax.experimental.pallas import tpu as pltpu
from jax.experimental.pallas import tpu_sc as plsc

N = 10000
D_IN = 128
D_HID = 128
D_EMB = 64
E = 320000

NC = 2
NS = 16
NW = NC * NS
K = 80
EPW = E // NW
NCHUNK = EPW // K
NPAD = 10240
RPT = NPAD // NS
ZROWS = 128
D = 128


def _make_sc_agg(with_deg):
    mesh = plsc.VectorSubcoreMesh(core_axis_name="c", subcore_axis_name="s")

    out_type = [jax.ShapeDtypeStruct((NC, NPAD, D), jnp.float32)]
    scratch = [
        pltpu.VMEM((K,), jnp.int32),
        pltpu.VMEM((K,), jnp.int32),
        pltpu.VMEM((K, D), jnp.float32),
        pltpu.VMEM((ZROWS, D), jnp.float32),
        pltpu.VMEM_SHARED((NPAD, D), jnp.float32),
        pltpu.SemaphoreType.DMA,
    ]
    if with_deg:
        out_type.append(jax.ShapeDtypeStruct((NW, NPAD), jnp.float32))
        scratch.insert(4, pltpu.VMEM((NPAD,), jnp.float32))

    @functools.partial(
        pl.kernel, mesh=mesh, out_type=out_type, scratch_types=scratch,
        compiler_params=pltpu.CompilerParams(needs_layout_passes=False))
    def agg(*refs):
        if with_deg:
            (y_hbm, src_hbm, dst_hbm, zero_hbm, out_hbm, deg_hbm,
             src_v, dst_v, rows_v, zbuf_v, deg_v, acc_sh, sem) = refs
        else:
            (y_hbm, src_hbm, dst_hbm, zero_hbm, out_hbm,
             src_v, dst_v, rows_v, zbuf_v, acc_sh, sem) = refs
        c = lax.axis_index("c")
        s = lax.axis_index("s")
        wid = c * NS + s

        pltpu.sync_copy(zero_hbm, zbuf_v)
        for kk in range(RPT // ZROWS):
            pltpu.sync_copy(
                zbuf_v, acc_sh.at[pl.ds(s * RPT + kk * ZROWS, ZROWS)])
        if with_deg:
            def zdeg(i, carry):
                deg_v[pl.ds(i * 16, 16)] = jnp.zeros((16,), jnp.float32)
                return carry
            lax.fori_loop(0, NPAD // 16, zdeg, 0)
        plsc.subcore_barrier()

        ones16 = jnp.ones((16,), jnp.float32)

        def body(g, carry):
            pltpu.sync_copy(src_hbm.at[wid, g], src_v)
            pltpu.sync_copy(dst_hbm.at[wid, g], dst_v)
            pltpu.async_copy(y_hbm.at[src_v], rows_v, sem).wait()
            pltpu.sync_copy(rows_v, acc_sh.at[dst_v], add=True)
            if with_deg:
                for j in range(K // 16):
                    idx16 = dst_v[pl.ds(j * 16, 16)]
                    plsc.addupdate_scatter(deg_v, [idx16], ones16)
            return carry

        lax.fori_loop(0, NCHUNK, body, 0)
        plsc.subcore_barrier()

        for kk in range(RPT // ZROWS):
            r0 = s * RPT + kk * ZROWS
            pltpu.sync_copy(acc_sh.at[pl.ds(r0, ZROWS)], zbuf_v)
            pltpu.sync_copy(zbuf_v, out_hbm.at[c, pl.ds(r0, ZROWS)])
        if with_deg:
            pltpu.sync_copy(deg_v, deg_hbm.at[wid])

    return agg


_sc_agg_deg = _make_sc_agg(True)
_sc_agg = _make_sc_agg(False)


def _tc_y1(x, wl):
    def body(x_ref, w_ref, o_ref):
        o_ref[...] = jnp.dot(x_ref[...], w_ref[...],
                             preferred_element_type=jnp.float32)

    return pl.pallas_call(
        body,
        out_shape=jax.ShapeDtypeStruct((N, D_HID), jnp.float32),
    )(x, wl)


def _tc_combine1(acc, deg, x, wr, b, wl2):
    def body(acc_ref, deg_ref, x_ref, wr_ref, b_ref, wl2_ref,
             h_ref, y_ref, inv_ref):
        agg = acc_ref[0, :N, :] + acc_ref[1, :N, :]
        onesw = jnp.ones((NW, 1), jnp.float32)
        degcol = lax.dot_general(
            deg_ref[...], onesw, (((0,), (0,)), ((), ())),
            preferred_element_type=jnp.float32)
        inv = 1.0 / jnp.maximum(degcol[:N, :], 1.0)
        h = jnp.maximum(
            agg * inv + b_ref[...]
            + jnp.dot(x_ref[...], wr_ref[...],
                      preferred_element_type=jnp.float32),
            0.0)
        h_ref[...] = h
        y_ref[...] = jnp.dot(h, wl2_ref[...],
                             preferred_element_type=jnp.float32)
        inv_ref[...] = inv

    return pl.pallas_call(
        body,
        out_shape=(
            jax.ShapeDtypeStruct((N, D_HID), jnp.float32),
            jax.ShapeDtypeStruct((N, D_HID), jnp.float32),
            jax.ShapeDtypeStruct((N, 1), jnp.float32),
        ),
    )(acc, deg, x, wr, b, wl2)


def _tc_combine2(acc, inv, h, wr, b):
    def body(acc_ref, inv_ref, h_ref, wr_ref, b_ref, ho_ref):
        agg = acc_ref[0, :N, :] + acc_ref[1, :N, :]
        ho_ref[...] = jnp.maximum(
            agg * inv_ref[...] + b_ref[...]
            + jnp.dot(h_ref[...], wr_ref[...],
                      preferred_element_type=jnp.float32),
            0.0)

    return pl.pallas_call(
        body,
        out_shape=jax.ShapeDtypeStruct((N, D_HID), jnp.float32),
    )(acc, inv, h, wr, b)


def _tc_final(acc, inv, h, wl, wr, b):
    def body(acc_ref, inv_ref, h_ref, wl_ref, wr_ref, b_ref, o_ref):
        agg = acc_ref[0, :N, :] + acc_ref[1, :N, :]
        mean = agg * inv_ref[...]
        o_ref[...] = (
            jnp.dot(mean, wl_ref[...], preferred_element_type=jnp.float32)
            + b_ref[...]
            + jnp.dot(h_ref[...], wr_ref[...],
                      preferred_element_type=jnp.float32))

    return pl.pallas_call(
        body,
        out_shape=jax.ShapeDtypeStruct((N, D_EMB), jnp.float32),
    )(acc, inv, h, wl, wr, b)


def kernel(x, edge_index, W1l, b1, W1r, W2l, b2, W2r, W3l, b3, W3r):
    ei = edge_index.astype(jnp.int32)
    src3 = ei[0].reshape(NW, NCHUNK, K)
    dst3 = ei[1].reshape(NW, NCHUNK, K)
    zrows = jnp.zeros((ZROWS, D), jnp.float32)

    y1 = _tc_y1(x, W1l)
    acc1, deg = _sc_agg_deg(y1, src3, dst3, zrows)
    h1, y2, inv = _tc_combine1(acc1, deg, x, W1r, b1.reshape(1, -1), W2l)
    acc2, = _sc_agg(y2, src3, dst3, zrows)
    h2 = _tc_combine2(acc2, inv, h1, W2r, b2.reshape(1, -1))
    acc3, = _sc_agg(h2, src3, dst3, zrows)
    return _tc_final(acc3, inv, h2, W3l, W3r, b3.reshape(1, -1))


# speedup vs baseline: 10.0095x; 1.9386x over previous
import functools

import jax
import jax.numpy as jnp
from jax import lax
from jax.experimental import pallas as pl
from jax.experimental.pallas import tpu as pltpu
from jax.experimental.pallas import tpu_sc as plsc

N = 10000
D_IN = 128
D_HID = 128
D_EMB = 64
E = 320000

NC = 2
NS = 16
NW = NC * NS
K = 40
EPW = E // NW
NCHUNK = EPW // K
NPAD = 10240
RPT = NPAD // NS
ZROWS = 40
D = 128


NBUF = 5
NT = NCHUNK // NBUF


def _make_sc_agg(with_deg):
    mesh = plsc.VectorSubcoreMesh(core_axis_name="c", subcore_axis_name="s")

    out_type = [jax.ShapeDtypeStruct((NC, NPAD, D), jnp.float32)]
    scratch = [pltpu.VMEM((NBUF * K,), jnp.int32) for _ in range(4)]
    scratch += [pltpu.VMEM((K,), jnp.int32) for _ in range(NBUF)]
    scratch += [pltpu.VMEM((K, D), jnp.float32) for _ in range(NBUF)]
    scratch += [pltpu.SemaphoreType.DMA, pltpu.SemaphoreType.DMA,
                pltpu.SemaphoreType.DMA,
                pltpu.VMEM_SHARED((NPAD, D), jnp.float32)]
    if with_deg:
        out_type.append(jax.ShapeDtypeStruct((NW, NPAD), jnp.float32))
        scratch.append(pltpu.VMEM((NPAD,), jnp.float32))

    @functools.partial(
        pl.kernel, mesh=mesh, out_type=out_type, scratch_types=scratch,
        compiler_params=pltpu.CompilerParams(needs_layout_passes=False))
    def agg(*refs):
        if with_deg:
            (y_hbm, src_hbm, dst_hbm, zero_hbm, out_hbm, deg_hbm,
             src_a, dst_a, src_b, dst_b, *rest,
             sem_i, sem_g, sem_s, acc_sh, deg_v) = refs
        else:
            (y_hbm, src_hbm, dst_hbm, zero_hbm, out_hbm,
             src_a, dst_a, src_b, dst_b, *rest,
             sem_i, sem_g, sem_s, acc_sh) = refs
        dstb = rest[:NBUF]
        rows = rest[NBUF:]
        c = lax.axis_index("c")
        s = lax.axis_index("s")
        wid = c * NS + s

        pltpu.sync_copy(zero_hbm, rows[0])
        for kk in range(RPT // ZROWS):
            pltpu.sync_copy(
                rows[0], acc_sh.at[pl.ds(s * RPT + kk * ZROWS, ZROWS)])
        if with_deg:
            def zdeg(i, carry):
                deg_v[pl.ds(i * 16, 16)] = jnp.zeros((16,), jnp.float32)
                return carry
            lax.fori_loop(0, NPAD // 16, zdeg, 0)
        plsc.subcore_barrier()

        ones16 = jnp.ones((16,), jnp.float32)

        def idx_copies(t, sb, db):
            sl = pl.ds(wid * EPW + t * (NBUF * K), NBUF * K)
            return (pltpu.make_async_copy(src_hbm.at[sl], sb, sem_i),
                    pltpu.make_async_copy(dst_hbm.at[sl], db, sem_i))

        tailmask = lax.iota(jnp.int32, 16) >= (32 - (K - 16))

        def hist(db):
            for b in range(NBUF):
                for j in range(K // 16):
                    idx16 = db[pl.ds(b * K + j * 16, 16)]
                    plsc.addupdate_scatter(deg_v, [idx16], ones16)
                idx16 = db[pl.ds(b * K + K - 16, 16)]
                plsc.addupdate_scatter(deg_v, [idx16], ones16, mask=tailmask)

        def process(sb, db):
            for b in range(NBUF):
                for off in (0, 16, K - 16):
                    dstb[b][pl.ds(off, 16)] = db[pl.ds(b * K + off, 16)]
            gs = [pltpu.async_copy(y_hbm.at[sb.at[pl.ds(b * K, K)]],
                                   rows[b], sem_g)
                  for b in range(NBUF)]
            if with_deg:
                hist(db)
            for h in gs:
                h.wait()
            ss = [pltpu.async_copy(rows[b], acc_sh.at[dstb[b]],
                                   sem_s, add=True)
                  for b in range(NBUF)]
            for h in ss:
                h.wait()

        for h in idx_copies(0, src_a, dst_a):
            h.start()

        def body(u, carry):
            t0 = 2 * u
            for h in idx_copies(t0, src_a, dst_a):
                h.wait()
            nxt = idx_copies(t0 + 1, src_b, dst_b)
            for h in nxt:
                h.start()
            process(src_a, dst_a)
            for h in nxt:
                h.wait()

            @pl.when(u < NT // 2 - 1)
            def _():
                for h in idx_copies(t0 + 2, src_a, dst_a):
                    h.start()

            process(src_b, dst_b)
            return carry

        lax.fori_loop(0, NT // 2, body, 0)
        plsc.subcore_barrier()

        stores = [None, None]
        for kk in range(RPT // ZROWS):
            b = kk % 2
            if stores[b] is not None:
                stores[b].wait()
            r0 = s * RPT + kk * ZROWS
            pltpu.async_copy(acc_sh.at[pl.ds(r0, ZROWS)], rows[b],
                             sem_g).wait()
            stores[b] = pltpu.async_copy(
                rows[b], out_hbm.at[c, pl.ds(r0, ZROWS)], sem_s)
        for st in stores:
            if st is not None:
                st.wait()
        if with_deg:
            pltpu.sync_copy(deg_v, deg_hbm.at[wid])

    return agg


_sc_agg_deg = _make_sc_agg(True)
_sc_agg = _make_sc_agg(False)


def _tc_y1(x, wl):
    def body(x_ref, w_ref, o_ref):
        o_ref[...] = jnp.dot(x_ref[...], w_ref[...],
                             preferred_element_type=jnp.float32)

    return pl.pallas_call(
        body,
        out_shape=jax.ShapeDtypeStruct((N, D_HID), jnp.float32),
    )(x, wl)


def _tc_combine1(acc, deg, x, wr, b, wl2):
    def body(acc_ref, deg_ref, x_ref, wr_ref, b_ref, wl2_ref,
             h_ref, y_ref, inv_ref):
        agg = acc_ref[0, :N, :] + acc_ref[1, :N, :]
        onesw = jnp.ones((NW, 1), jnp.float32)
        degcol = lax.dot_general(
            deg_ref[...], onesw, (((0,), (0,)), ((), ())),
            preferred_element_type=jnp.float32)
        inv = 1.0 / jnp.maximum(degcol[:N, :], 1.0)
        h = jnp.maximum(
            agg * inv + b_ref[...]
            + jnp.dot(x_ref[...], wr_ref[...],
                      preferred_element_type=jnp.float32),
            0.0)
        h_ref[...] = h
        y_ref[...] = jnp.dot(h, wl2_ref[...],
                             preferred_element_type=jnp.float32)
        inv_ref[...] = inv

    return pl.pallas_call(
        body,
        out_shape=(
            jax.ShapeDtypeStruct((N, D_HID), jnp.float32),
            jax.ShapeDtypeStruct((N, D_HID), jnp.float32),
            jax.ShapeDtypeStruct((N, 1), jnp.float32),
        ),
    )(acc, deg, x, wr, b, wl2)


def _tc_combine2(acc, inv, h, wr, b):
    def body(acc_ref, inv_ref, h_ref, wr_ref, b_ref, ho_ref):
        agg = acc_ref[0, :N, :] + acc_ref[1, :N, :]
        ho_ref[...] = jnp.maximum(
            agg * inv_ref[...] + b_ref[...]
            + jnp.dot(h_ref[...], wr_ref[...],
                      preferred_element_type=jnp.float32),
            0.0)

    return pl.pallas_call(
        body,
        out_shape=jax.ShapeDtypeStruct((N, D_HID), jnp.float32),
    )(acc, inv, h, wr, b)


def _tc_final(acc, inv, h, wl, wr, b):
    def body(acc_ref, inv_ref, h_ref, wl_ref, wr_ref, b_ref, o_ref):
        agg = acc_ref[0, :N, :] + acc_ref[1, :N, :]
        mean = agg * inv_ref[...]
        o_ref[...] = (
            jnp.dot(mean, wl_ref[...], preferred_element_type=jnp.float32)
            + b_ref[...]
            + jnp.dot(h_ref[...], wr_ref[...],
                      preferred_element_type=jnp.float32))

    return pl.pallas_call(
        body,
        out_shape=jax.ShapeDtypeStruct((N, D_EMB), jnp.float32),
    )(acc, inv, h, wl, wr, b)


def kernel(x, edge_index, W1l, b1, W1r, W2l, b2, W2r, W3l, b3, W3r):
    ei = edge_index.astype(jnp.int32)
    src3 = ei[0]
    dst3 = ei[1]
    zrows = jnp.zeros((ZROWS, D), jnp.float32)

    y1 = _tc_y1(x, W1l)
    acc1, deg = _sc_agg_deg(y1, src3, dst3, zrows)
    h1, y2, inv = _tc_combine1(acc1, deg, x, W1r, b1.reshape(1, -1), W2l)
    acc2, = _sc_agg(y2, src3, dst3, zrows)
    h2 = _tc_combine2(acc2, inv, h1, W2r, b2.reshape(1, -1))
    acc3, = _sc_agg(h2, src3, dst3, zrows)
    return _tc_final(acc3, inv, h2, W3l, W3r, b3.reshape(1, -1))


# speedup vs baseline: 11.7562x; 1.1745x over previous
import functools

import jax
import jax.numpy as jnp
from jax import lax
from jax.experimental import pallas as pl
from jax.experimental.pallas import tpu as pltpu
from jax.experimental.pallas import tpu_sc as plsc

N = 10000
D_IN = 128
D_HID = 128
D_EMB = 64
E = 320000

NC = 2
NS = 16
NW = NC * NS
K = 40
EPW = E // NW
NCHUNK = EPW // K
NPAD = 10240
RPT = NPAD // NS
ZROWS = RPT
D = 128


NBUF = 5
NT = NCHUNK // NBUF


def _make_sc_agg(with_deg):
    mesh = plsc.VectorSubcoreMesh(core_axis_name="c", subcore_axis_name="s")

    out_type = [jax.ShapeDtypeStruct((NC, NPAD, D), jnp.float32)]
    scratch = [pltpu.VMEM((NBUF * K,), jnp.int32) for _ in range(4)]
    scratch += [pltpu.VMEM((K,), jnp.int32) for _ in range(NBUF)]
    scratch += [pltpu.VMEM((K, D), jnp.float32) for _ in range(NBUF)]
    scratch += [pltpu.SemaphoreType.DMA, pltpu.SemaphoreType.DMA,
                pltpu.SemaphoreType.DMA,
                pltpu.VMEM_SHARED((NPAD, D), jnp.float32)]
    if with_deg:
        out_type.append(jax.ShapeDtypeStruct((NW, NPAD), jnp.float32))
        scratch.append(pltpu.VMEM((NPAD,), jnp.float32))

    @functools.partial(
        pl.kernel, mesh=mesh, out_type=out_type, scratch_types=scratch,
        compiler_params=pltpu.CompilerParams(needs_layout_passes=False))
    def agg(*refs):
        if with_deg:
            (y_hbm, src_hbm, dst_hbm, zero_hbm, out_hbm, deg_hbm,
             src_a, dst_a, src_b, dst_b, *rest,
             sem_i, sem_g, sem_s, acc_sh, deg_v) = refs
        else:
            (y_hbm, src_hbm, dst_hbm, zero_hbm, out_hbm,
             src_a, dst_a, src_b, dst_b, *rest,
             sem_i, sem_g, sem_s, acc_sh) = refs
        dstb = rest[:NBUF]
        rows = rest[NBUF:]
        c = lax.axis_index("c")
        s = lax.axis_index("s")
        wid = c * NS + s

        pltpu.sync_copy(zero_hbm, acc_sh.at[pl.ds(s * RPT, RPT)])
        if with_deg:
            def zdeg(i, carry):
                deg_v[pl.ds(i * 16, 16)] = jnp.zeros((16,), jnp.float32)
                return carry
            lax.fori_loop(0, NPAD // 16, zdeg, 0)
        plsc.subcore_barrier()

        ones16 = jnp.ones((16,), jnp.float32)

        def idx_copies(t, sb, db):
            sl = pl.ds(wid * EPW + t * (NBUF * K), NBUF * K)
            return (pltpu.make_async_copy(src_hbm.at[sl], sb, sem_i),
                    pltpu.make_async_copy(dst_hbm.at[sl], db, sem_i))

        tailmask = lax.iota(jnp.int32, 16) >= ((K // 16) * 16 - (K - 16))

        def hist(db):
            for b in range(NBUF):
                for j in range(K // 16):
                    idx16 = db[pl.ds(b * K + j * 16, 16)]
                    plsc.addupdate_scatter(deg_v, [idx16], ones16)
                idx16 = db[pl.ds(b * K + K - 16, 16)]
                plsc.addupdate_scatter(deg_v, [idx16], ones16, mask=tailmask)

        def drain_scatters():
            for b in range(NBUF):
                pltpu.make_async_copy(rows[b], acc_sh.at[dstb[b]],
                                      sem_s).wait()

        def process(sb, db):
            for b in range(NBUF):
                for off in list(range(0, K - 16, 16)) + [K - 16]:
                    dstb[b][pl.ds(off, 16)] = db[pl.ds(b * K + off, 16)]
            gs = [pltpu.async_copy(y_hbm.at[sb.at[pl.ds(b * K, K)]],
                                   rows[b], sem_g)
                  for b in range(NBUF)]
            if with_deg:
                hist(db)
            for b in range(NBUF):
                gs[b].wait()
                pltpu.async_copy(rows[b], acc_sh.at[dstb[b]],
                                 sem_s, add=True)

        for h in idx_copies(0, src_a, dst_a):
            h.start()

        def body(u, carry):
            t0 = 2 * u
            for h in idx_copies(t0, src_a, dst_a):
                h.wait()
            nxt = idx_copies(t0 + 1, src_b, dst_b)
            for h in nxt:
                h.start()

            @pl.when(u > 0)
            def _():
                drain_scatters()

            process(src_a, dst_a)
            for h in nxt:
                h.wait()

            @pl.when(u < NT // 2 - 1)
            def _():
                for h in idx_copies(t0 + 2, src_a, dst_a):
                    h.start()

            drain_scatters()
            process(src_b, dst_b)
            return carry

        lax.fori_loop(0, NT // 2, body, 0)
        drain_scatters()
        plsc.subcore_barrier()

        r0 = s * RPT
        pltpu.async_copy(acc_sh.at[pl.ds(r0, RPT)],
                         out_hbm.at[c, pl.ds(r0, RPT)], sem_s).wait()
        if with_deg:
            pltpu.sync_copy(deg_v, deg_hbm.at[wid])

    return agg


_sc_agg_deg = _make_sc_agg(True)
_sc_agg = _make_sc_agg(False)


def _tc_y1(x, wl):
    def body(x_ref, w_ref, o_ref):
        o_ref[...] = jnp.dot(x_ref[...], w_ref[...],
                             preferred_element_type=jnp.float32)

    return pl.pallas_call(
        body,
        out_shape=jax.ShapeDtypeStruct((N, D_HID), jnp.float32),
    )(x, wl)


def _tc_combine1(acc, deg, x, wr, b, wl2):
    def body(acc_ref, deg_ref, x_ref, wr_ref, b_ref, wl2_ref,
             h_ref, y_ref, inv_ref):
        agg = acc_ref[0, :N, :] + acc_ref[1, :N, :]
        onesw = jnp.ones((NW, 1), jnp.float32)
        degcol = lax.dot_general(
            deg_ref[...], onesw, (((0,), (0,)), ((), ())),
            preferred_element_type=jnp.float32)
        inv = 1.0 / jnp.maximum(degcol[:N, :], 1.0)
        h = jnp.maximum(
            agg * inv + b_ref[...]
            + jnp.dot(x_ref[...], wr_ref[...],
                      preferred_element_type=jnp.float32),
            0.0)
        h_ref[...] = h
        y_ref[...] = jnp.dot(h, wl2_ref[...],
                             preferred_element_type=jnp.float32)
        inv_ref[...] = inv

    return pl.pallas_call(
        body,
        out_shape=(
            jax.ShapeDtypeStruct((N, D_HID), jnp.float32),
            jax.ShapeDtypeStruct((N, D_HID), jnp.float32),
            jax.ShapeDtypeStruct((N, 1), jnp.float32),
        ),
    )(acc, deg, x, wr, b, wl2)


def _tc_combine2(acc, inv, h, wr, b):
    def body(acc_ref, inv_ref, h_ref, wr_ref, b_ref, ho_ref):
        agg = acc_ref[0, :N, :] + acc_ref[1, :N, :]
        ho_ref[...] = jnp.maximum(
            agg * inv_ref[...] + b_ref[...]
            + jnp.dot(h_ref[...], wr_ref[...],
                      preferred_element_type=jnp.float32),
            0.0)

    return pl.pallas_call(
        body,
        out_shape=jax.ShapeDtypeStruct((N, D_HID), jnp.float32),
    )(acc, inv, h, wr, b)


def _tc_final(acc, inv, h, wl, wr, b):
    def body(acc_ref, inv_ref, h_ref, wl_ref, wr_ref, b_ref, o_ref):
        agg = acc_ref[0, :N, :] + acc_ref[1, :N, :]
        mean = agg * inv_ref[...]
        o_ref[...] = (
            jnp.dot(mean, wl_ref[...], preferred_element_type=jnp.float32)
            + b_ref[...]
            + jnp.dot(h_ref[...], wr_ref[...],
                      preferred_element_type=jnp.float32))

    return pl.pallas_call(
        body,
        out_shape=jax.ShapeDtypeStruct((N, D_EMB), jnp.float32),
    )(acc, inv, h, wl, wr, b)


def kernel(x, edge_index, W1l, b1, W1r, W2l, b2, W2r, W3l, b3, W3r):
    ei = edge_index.astype(jnp.int32)
    src3 = ei[0]
    dst3 = ei[1]
    zrows = jnp.zeros((ZROWS, D), jnp.float32)

    y1 = _tc_y1(x, W1l)
    acc1, deg = _sc_agg_deg(y1, src3, dst3, zrows)
    h1, y2, inv = _tc_combine1(acc1, deg, x, W1r, b1.reshape(1, -1), W2l)
    acc2, = _sc_agg(y2, src3, dst3, zrows)
    h2 = _tc_combine2(acc2, inv, h1, W2r, b2.reshape(1, -1))
    acc3, = _sc_agg(h2, src3, dst3, zrows)
    return _tc_final(acc3, inv, h2, W3l, W3r, b3.reshape(1, -1))
